# baseline (speedup 1.0000x reference)
import functools

import jax
import jax.numpy as jnp
from jax import lax
from jax.experimental import pallas as pl
from jax.experimental.pallas import tpu as pltpu
from jax.experimental.pallas import tpu_sc as plsc

N_NODES_ = 10000
N_PAD = 10240
L_SC = 10048
N_EDGES_ = 320000
N_COMS_ = 4
COM_DIM_ = 32
EPS = 1e-5
BLK = 1024
GRID = N_PAD // BLK
NC, NS = 2, 16
NW = NC * NS
F32 = jnp.float32


def _stage0(xp, W_enc, b_enc_row):
    def body(x_ref, we_ref, be_ref, enc_ref, xT_ref):
        xb = x_ref[...]
        enc_ref[...] = jnp.dot(xb, we_ref[...],
                               preferred_element_type=F32) + be_ref[...]
        xT_ref[...] = xb.T

    return pl.pallas_call(
        body,
        grid=(GRID,),
        in_specs=[
            pl.BlockSpec((BLK, 128), lambda i: (i, 0)),
            pl.BlockSpec((128, 128), lambda i: (0, 0)),
            pl.BlockSpec((1, 128), lambda i: (0, 0)),
        ],
        out_specs=[
            pl.BlockSpec((BLK, 128), lambda i: (i, 0)),
            pl.BlockSpec((128, BLK), lambda i: (0, i)),
        ],
        out_shape=[
            jax.ShapeDtypeStruct((N_PAD, 128), F32),
            jax.ShapeDtypeStruct((128, N_PAD), F32),
        ],
    )(xp, W_enc, b_enc_row)


def _stage1(xp, agg0a, agg0b, W0a, b0a_row, W0b, b0b_row):
    def body(x_ref, aa_ref, ab_ref, wa_ref, ba_ref, wb_ref, bb_ref,
             hpre_ref, st_ref):
        i = pl.program_id(0)
        xb = x_ref[...]
        wa = wa_ref[...]
        wb = wb_ref[...]
        outs = []
        for k in range(N_COMS_):
            agg_ref = aa_ref if k < 2 else ab_ref
            c2 = k % 2
            aggb = (agg_ref[c2 * 128:(c2 + 1) * 128, :]).T
            t = jnp.maximum(
                jnp.dot(xb + aggb, wa, preferred_element_type=F32)
                + ba_ref[...], 0.0)
            outs.append(jnp.dot(t, wb, preferred_element_type=F32)
                        + bb_ref[...])
        hpre = jnp.concatenate(outs, axis=1)
        hpre_ref[...] = hpre
        row = i * BLK + lax.broadcasted_iota(jnp.int32, (BLK, 128), 0)
        hm = jnp.where(row < N_NODES_, hpre, 0.0)
        s = jnp.sum(hm, axis=0, keepdims=True)
        ss = jnp.sum(hm * hm, axis=0, keepdims=True)

        @pl.when(i == 0)
        def _():
            st_ref[...] = jnp.zeros_like(st_ref)
        st_ref[...] += jnp.concatenate([s, ss], axis=0)

    return pl.pallas_call(
        body,
        grid=(GRID,),
        in_specs=[
            pl.BlockSpec((BLK, 128), lambda i: (i, 0)),
            pl.BlockSpec((256, BLK), lambda i: (0, i)),
            pl.BlockSpec((256, BLK), lambda i: (0, i)),
            pl.BlockSpec((128, COM_DIM_), lambda i: (0, 0)),
            pl.BlockSpec((1, COM_DIM_), lambda i: (0, 0)),
            pl.BlockSpec((COM_DIM_, COM_DIM_), lambda i: (0, 0)),
            pl.BlockSpec((1, COM_DIM_), lambda i: (0, 0)),
        ],
        out_specs=[
            pl.BlockSpec((BLK, 128), lambda i: (i, 0)),
            pl.BlockSpec((2, 128), lambda i: (0, 0)),
        ],
        out_shape=[
            jax.ShapeDtypeStruct((N_PAD, 128), F32),
            jax.ShapeDtypeStruct((2, 128), F32),
        ],
    )(xp, agg0a, agg0b, W0a, b0a_row, W0b, b0b_row)


def _stage_var_row(hpre, st):
    def body(hpre_ref, st_ref, var_ref):
        i = pl.program_id(0)
        mu = st_ref[...][0:1, :] / N_NODES_
        row = i * BLK + lax.broadcasted_iota(jnp.int32, (BLK, 128), 0)
        dv = jnp.where(row < N_NODES_, hpre_ref[...] - mu, 0.0)
        s = jnp.sum(dv * dv, axis=0, keepdims=True)

        @pl.when(i == 0)
        def _():
            var_ref[...] = jnp.zeros_like(var_ref)
        var_ref[...] += s

    return pl.pallas_call(
        body,
        grid=(GRID,),
        in_specs=[
            pl.BlockSpec((BLK, 128), lambda i: (i, 0)),
            pl.BlockSpec((2, 128), lambda i: (0, 0)),
        ],
        out_specs=pl.BlockSpec((1, 128), lambda i: (0, 0)),
        out_shape=jax.ShapeDtypeStruct((1, 128), F32),
    )(hpre, st)


def _stage_post_row(hpre, st, varsum, g_row, be_row):
    def body(hpre_ref, st_ref, var_ref, g_ref, be_ref, out_ref, hT_ref):
        mu = st_ref[...][0:1, :] / N_NODES_
        var = var_ref[...] / N_NODES_
        h = jnp.maximum(
            (hpre_ref[...] - mu) / jnp.sqrt(var + EPS) * g_ref[...]
            + be_ref[...], 0.0)
        out_ref[...] = h
        hT_ref[...] = h.T

    return pl.pallas_call(
        body,
        grid=(GRID,),
        in_specs=[
            pl.BlockSpec((BLK, 128), lambda i: (i, 0)),
            pl.BlockSpec((2, 128), lambda i: (0, 0)),
            pl.BlockSpec((1, 128), lambda i: (0, 0)),
            pl.BlockSpec((1, 128), lambda i: (0, 0)),
            pl.BlockSpec((1, 128), lambda i: (0, 0)),
        ],
        out_specs=[
            pl.BlockSpec((BLK, 128), lambda i: (i, 0)),
            pl.BlockSpec((128, BLK), lambda i: (0, i)),
        ],
        out_shape=[
            jax.ShapeDtypeStruct((N_PAD, 128), F32),
            jax.ShapeDtypeStruct((128, N_PAD), F32),
        ],
    )(hpre, st, varsum, g_row, be_row)


def _stage_pre2(h_T, agg_T, Wa, ba_t, Wb, bb_t):
    def body(h_ref, agg_ref, wa_ref, ba_ref, wb_ref, bb_ref, hpre_ref, st_ref):
        i = pl.program_id(0)
        s_in = h_ref[...] + agg_ref[...]
        wa = wa_ref[...]
        wb = wb_ref[...]
        outs = []
        for k in range(N_COMS_):
            u = lax.dot_general(wa, s_in[k * COM_DIM_:(k + 1) * COM_DIM_, :],
                                (((0,), (0,)), ((), ())),
                                preferred_element_type=F32)
            u = jnp.maximum(u + ba_ref[0:COM_DIM_, :], 0.0)
            outs.append(lax.dot_general(wb, u, (((0,), (0,)), ((), ())),
                                        preferred_element_type=F32))
        hpre = jnp.concatenate(outs, axis=0) + bb_ref[...]
        hpre_ref[...] = hpre
        col = i * BLK + lax.broadcasted_iota(jnp.int32, (128, BLK), 1)
        hm = jnp.where(col < N_NODES_, hpre, 0.0)
        su = jnp.sum(hm, axis=1, keepdims=True)
        ss = jnp.sum(hm * hm, axis=1, keepdims=True)

        @pl.when(i == 0)
        def _():
            st_ref[...] = jnp.zeros_like(st_ref)
        st_ref[...] += jnp.concatenate([su, ss], axis=1)

    return pl.pallas_call(
        body,
        grid=(GRID,),
        in_specs=[
            pl.BlockSpec((128, BLK), lambda i: (0, i)),
            pl.BlockSpec((128, BLK), lambda i: (0, i)),
            pl.BlockSpec((COM_DIM_, COM_DIM_), lambda i: (0, 0)),
            pl.BlockSpec((128, 1), lambda i: (0, 0)),
            pl.BlockSpec((COM_DIM_, COM_DIM_), lambda i: (0, 0)),
            pl.BlockSpec((128, 1), lambda i: (0, 0)),
        ],
        out_specs=[
            pl.BlockSpec((128, BLK), lambda i: (0, i)),
            pl.BlockSpec((128, 2), lambda i: (0, 0)),
        ],
        out_shape=[
            jax.ShapeDtypeStruct((128, N_PAD), F32),
            jax.ShapeDtypeStruct((128, 2), F32),
        ],
    )(h_T, agg_T, Wa, ba_t, Wb, bb_t)


def _stage_var(hpre_T, st):
    def body(hpre_ref, st_ref, var_ref):
        i = pl.program_id(0)
        mu = st_ref[...][:, 0:1] / N_NODES_
        col = i * BLK + lax.broadcasted_iota(jnp.int32, (128, BLK), 1)
        dv = jnp.where(col < N_NODES_, hpre_ref[...] - mu, 0.0)
        s = jnp.sum(dv * dv, axis=1, keepdims=True)

        @pl.when(i == 0)
        def _():
            var_ref[...] = jnp.zeros_like(var_ref)
        var_ref[...] += s

    return pl.pallas_call(
        body,
        grid=(GRID,),
        in_specs=[
            pl.BlockSpec((128, BLK), lambda i: (0, i)),
            pl.BlockSpec((128, 2), lambda i: (0, 0)),
        ],
        out_specs=pl.BlockSpec((128, 1), lambda i: (0, 0)),
        out_shape=jax.ShapeDtypeStruct((128, 1), F32),
    )(hpre_T, st)


def _stage_post(hpre_T, st, varsum, g_t, be_t):
    def body(hpre_ref, st_ref, var_ref, g_ref, be_ref, out_ref):
        mu = st_ref[...][:, 0:1] / N_NODES_
        var = var_ref[...] / N_NODES_
        h = jnp.maximum(
            (hpre_ref[...] - mu) / jnp.sqrt(var + EPS) * g_ref[...]
            + be_ref[...], 0.0)
        out_ref[...] = h.T

    return pl.pallas_call(
        body,
        grid=(GRID,),
        in_specs=[
            pl.BlockSpec((128, BLK), lambda i: (0, i)),
            pl.BlockSpec((128, 2), lambda i: (0, 0)),
            pl.BlockSpec((128, 1), lambda i: (0, 0)),
            pl.BlockSpec((128, 1), lambda i: (0, 0)),
            pl.BlockSpec((128, 1), lambda i: (0, 0)),
        ],
        out_specs=pl.BlockSpec((BLK, 128), lambda i: (i, 0)),
        out_shape=jax.ShapeDtypeStruct((N_PAD, 128), F32),
    )(hpre_T, st, varsum, g_t, be_t)


def _sc_mesh():
    return plsc.VectorSubcoreMesh(core_axis_name="c", subcore_axis_name="s",
                                  num_cores=NC, num_subcores=NS)


def _zero_accs(accs):
    @pl.loop(0, L_SC // 16)
    def _(i):
        z = jnp.zeros((16,), F32)
        for a in accs:
            a[pl.ds(i * 16, 16)] = z


def _edge_pipeline(src_hbm, dst_hbm, w_hbm, w_offs, sbufs, dbufs, wbufs,
                   sems, ec, process):
    n_chunks = N_EDGES_ // ec

    def start(g, b):
        base = g * ec
        pltpu.async_copy(src_hbm.at[pl.ds(base, ec)], sbufs[b], sems[b])
        pltpu.async_copy(dst_hbm.at[pl.ds(base, ec)], dbufs[b], sems[b])
        for t, off in enumerate(w_offs):
            pltpu.async_copy(w_hbm.at[pl.ds(off + base, ec)], wbufs[t][b],
                             sems[b])

    def wait(b):
        pltpu.make_async_copy(src_hbm.at[pl.ds(0, ec)], sbufs[b],
                              sems[b]).wait()
        pltpu.make_async_copy(dst_hbm.at[pl.ds(0, ec)], dbufs[b],
                              sems[b]).wait()
        for t in range(len(w_offs)):
            pltpu.make_async_copy(w_hbm.at[pl.ds(0, ec)], wbufs[t][b],
                                  sems[b]).wait()

    start(0, 0)

    @pl.loop(0, n_chunks, step=2)
    def _(g):
        start(g + 1, 1)
        wait(0)
        process(0)

        @pl.when(g + 2 < n_chunks)
        def _():
            start(g + 2, 0)
        wait(1)
        process(1)


def _make_sc_full():
    EC = 1000
    scratch = (
        [pltpu.VMEM((L_SC,), F32) for _ in range(4)]
        + [pltpu.VMEM((L_SC,), F32) for _ in range(8)]
        + [pltpu.VMEM((EC,), jnp.int32) for _ in range(4)]
        + [pltpu.VMEM((EC,), F32) for _ in range(4)]
        + [pltpu.SemaphoreType.DMA, pltpu.SemaphoreType.DMA]
    )

    @functools.partial(
        pl.kernel,
        out_type=jax.ShapeDtypeStruct((256 * N_PAD,), F32),
        mesh=_sc_mesh(),
        scratch_types=scratch,
        compiler_params=pltpu.CompilerParams(needs_layout_passes=False),
    )
    def seg(xT_hbm, src_hbm, dst_hbm, w2_hbm, agg_hbm, *refs):
        tabs = refs[0:4]
        accs = refs[4:12]
        sbufs = refs[12:14]
        dbufs = refs[14:16]
        wb = refs[16:20]
        wbufs = [wb[0:2], wb[2:4]]
        sems = refs[20:22]

        wid = lax.axis_index("s") * NC + lax.axis_index("c")

        for j in range(4):
            off = pl.multiple_of((4 * wid + j) * N_PAD, 8)
            pltpu.sync_copy(xT_hbm.at[pl.ds(off, L_SC)], tabs[j])
        _zero_accs(accs)

        def process(b):
            sb, db = sbufs[b], dbufs[b]

            @plsc.parallel_loop(0, EC // 16, unroll=8)
            def _(i):
                off = i * 16
                sidx = sb[pl.ds(off, 16)]
                didx = db[pl.ds(off, 16)]
                w0 = wbufs[0][b][pl.ds(off, 16)]
                w1 = wbufs[1][b][pl.ds(off, 16)]
                for j in range(4):
                    v = plsc.load_gather(tabs[j], [sidx])
                    plsc.addupdate_scatter(accs[j], [didx], v * w0)
                    plsc.addupdate_scatter(accs[4 + j], [didx], v * w1)

        _edge_pipeline(src_hbm, dst_hbm, w2_hbm, (0, N_EDGES_),
                       sbufs, dbufs, wbufs, sems, EC, process)

        for c2 in range(2):
            for j in range(4):
                off = pl.multiple_of((c2 * 128 + 4 * wid + j) * N_PAD, 8)
                pltpu.sync_copy(accs[4 * c2 + j],
                                agg_hbm.at[pl.ds(off, L_SC)])

    return seg


def _make_sc_perk():
    EC = 2000
    scratch = (
        [pltpu.VMEM((L_SC,), F32) for _ in range(4)]
        + [pltpu.VMEM((L_SC,), F32) for _ in range(4)]
        + [pltpu.VMEM((EC,), jnp.int32) for _ in range(4)]
        + [pltpu.VMEM((EC,), F32) for _ in range(8)]
        + [pltpu.SemaphoreType.DMA, pltpu.SemaphoreType.DMA]
    )

    @functools.partial(
        pl.kernel,
        out_type=jax.ShapeDtypeStruct((128 * N_PAD,), F32),
        mesh=_sc_mesh(),
        scratch_types=scratch,
        compiler_params=pltpu.CompilerParams(needs_layout_passes=False),
    )
    def seg(hT_hbm, src_hbm, dst_hbm, w_hbm, agg_hbm, *refs):
        tabs = refs[0:4]
        accs = refs[4:8]
        sbufs = refs[8:10]
        dbufs = refs[10:12]
        wb = refs[12:20]
        wbufs = [wb[0:2], wb[2:4], wb[4:6], wb[6:8]]
        sems = refs[20:22]

        wid = lax.axis_index("s") * NC + lax.axis_index("c")

        for k in range(N_COMS_):
            off = pl.multiple_of((k * COM_DIM_ + wid) * N_PAD, 8)
            pltpu.sync_copy(hT_hbm.at[pl.ds(off, L_SC)], tabs[k])
        _zero_accs(accs)

        def process(b):
            sb, db = sbufs[b], dbufs[b]

            @plsc.parallel_loop(0, EC // 16, unroll=8)
            def _(i):
                off = i * 16
                sidx = sb[pl.ds(off, 16)]
                didx = db[pl.ds(off, 16)]
                for k in range(N_COMS_):
                    v = plsc.load_gather(tabs[k], [sidx])
                    wk = wbufs[k][b][pl.ds(off, 16)]
                    plsc.addupdate_scatter(accs[k], [didx], v * wk)

        _edge_pipeline(src_hbm, dst_hbm, w_hbm,
                       tuple(k * N_EDGES_ for k in range(N_COMS_)),
                       sbufs, dbufs, wbufs, sems, EC, process)

        for k in range(N_COMS_):
            off = pl.multiple_of((k * COM_DIM_ + wid) * N_PAD, 8)
            pltpu.sync_copy(accs[k], agg_hbm.at[pl.ds(off, L_SC)])

    return seg


_sc_seg_full = _make_sc_full()
_sc_seg_perk = _make_sc_perk()


def kernel(x, edge_index, edge_weight_list, W_enc, b_enc,
           W0a, b0a, W0b, b0b, g0, be0,
           W1a, b1a, W1b, b1b, g1, be1):
    src = edge_index[0].astype(jnp.int32)
    dst = edge_index[1].astype(jnp.int32)
    wflat = edge_weight_list.astype(F32).reshape(-1)
    xp = jnp.pad(x.astype(F32), ((0, N_PAD - N_NODES_), (0, 0)))

    enc_p, xT = _stage0(xp, W_enc, b_enc.reshape(1, -1))

    xT_flat = xT.reshape(-1)
    agg0a = _sc_seg_full(xT_flat, src, dst,
                         wflat[:2 * N_EDGES_]).reshape(256, N_PAD)
    agg0b = _sc_seg_full(xT_flat, src, dst,
                         wflat[2 * N_EDGES_:]).reshape(256, N_PAD)
    h1pre, st1 = _stage1(xp, agg0a, agg0b, W0a, b0a.reshape(1, -1),
                         W0b, b0b.reshape(1, -1))
    vs1 = _stage_var_row(h1pre, st1)
    out1_p, h1T = _stage_post_row(h1pre, st1, vs1,
                                  jnp.tile(g0, N_COMS_).reshape(1, -1),
                                  jnp.tile(be0, N_COMS_).reshape(1, -1))

    agg1 = _sc_seg_perk(h1T.reshape(-1), src, dst,
                        wflat).reshape(128, N_PAD)
    h2pre, st2 = _stage_pre2(h1T, agg1, W1a,
                             jnp.tile(b1a, N_COMS_).reshape(-1, 1),
                             W1b, jnp.tile(b1b, N_COMS_).reshape(-1, 1))
    vs2 = _stage_var(h2pre, st2)
    out2_p = _stage_post(h2pre, st2, vs2,
                         jnp.tile(g1, N_COMS_).reshape(-1, 1),
                         jnp.tile(be1, N_COMS_).reshape(-1, 1))

    return (enc_p[:N_NODES_], out1_p[:N_NODES_], out2_p[:N_NODES_])


# speedup vs baseline: 1.0262x; 1.0262x over previous
import functools

import jax
import jax.numpy as jnp
from jax import lax
from jax.experimental import pallas as pl
from jax.experimental.pallas import tpu as pltpu
from jax.experimental.pallas import tpu_sc as plsc

N_NODES_ = 10000
N_PAD = 10240
L_SC = 10048
N_EDGES_ = 320000
N_COMS_ = 4
COM_DIM_ = 32
EPS = 1e-5
BLK = 1024
GRID = N_PAD // BLK
NC, NS = 2, 16
NW = NC * NS
F32 = jnp.float32


def _stage0(xp, W_enc, b_enc_row):
    def body(x_ref, we_ref, be_ref, enc_ref, xT_ref):
        xb = x_ref[...]
        enc_ref[...] = jnp.dot(xb, we_ref[...],
                               preferred_element_type=F32) + be_ref[...]
        xT_ref[...] = xb.T

    return pl.pallas_call(
        body,
        grid=(GRID,),
        in_specs=[
            pl.BlockSpec((BLK, 128), lambda i: (i, 0)),
            pl.BlockSpec((128, 128), lambda i: (0, 0)),
            pl.BlockSpec((1, 128), lambda i: (0, 0)),
        ],
        out_specs=[
            pl.BlockSpec((BLK, 128), lambda i: (i, 0)),
            pl.BlockSpec((128, BLK), lambda i: (0, i)),
        ],
        out_shape=[
            jax.ShapeDtypeStruct((N_PAD, 128), F32),
            jax.ShapeDtypeStruct((128, N_PAD), F32),
        ],
    )(xp, W_enc, b_enc_row)


def _stage1(xp, agg0a, agg0b, W0a, b0a_row, W0b, b0b_row):
    def body(x_ref, aa_ref, ab_ref, wa_ref, ba_ref, wb_ref, bb_ref,
             hpre_ref, st_ref):
        i = pl.program_id(0)
        xb = x_ref[...]
        wa = wa_ref[...]
        wb = wb_ref[...]
        outs = []
        for k in range(N_COMS_):
            agg_ref = aa_ref if k < 2 else ab_ref
            c2 = k % 2
            aggb = (agg_ref[c2 * 128:(c2 + 1) * 128, :]).T
            t = jnp.maximum(
                jnp.dot(xb + aggb, wa, preferred_element_type=F32)
                + ba_ref[...], 0.0)
            outs.append(jnp.dot(t, wb, preferred_element_type=F32)
                        + bb_ref[...])
        hpre = jnp.concatenate(outs, axis=1)
        hpre_ref[...] = hpre
        row = i * BLK + lax.broadcasted_iota(jnp.int32, (BLK, 128), 0)
        hm = jnp.where(row < N_NODES_, hpre, 0.0)
        s = jnp.sum(hm, axis=0, keepdims=True)
        ss = jnp.sum(hm * hm, axis=0, keepdims=True)

        @pl.when(i == 0)
        def _():
            st_ref[...] = jnp.zeros_like(st_ref)
        st_ref[...] += jnp.concatenate([s, ss], axis=0)

    return pl.pallas_call(
        body,
        grid=(GRID,),
        in_specs=[
            pl.BlockSpec((BLK, 128), lambda i: (i, 0)),
            pl.BlockSpec((256, BLK), lambda i: (0, i)),
            pl.BlockSpec((256, BLK), lambda i: (0, i)),
            pl.BlockSpec((128, COM_DIM_), lambda i: (0, 0)),
            pl.BlockSpec((1, COM_DIM_), lambda i: (0, 0)),
            pl.BlockSpec((COM_DIM_, COM_DIM_), lambda i: (0, 0)),
            pl.BlockSpec((1, COM_DIM_), lambda i: (0, 0)),
        ],
        out_specs=[
            pl.BlockSpec((BLK, 128), lambda i: (i, 0)),
            pl.BlockSpec((2, 128), lambda i: (0, 0)),
        ],
        out_shape=[
            jax.ShapeDtypeStruct((N_PAD, 128), F32),
            jax.ShapeDtypeStruct((2, 128), F32),
        ],
    )(xp, agg0a, agg0b, W0a, b0a_row, W0b, b0b_row)


def _stage_var_row(hpre, st):
    def body(hpre_ref, st_ref, var_ref):
        i = pl.program_id(0)
        mu = st_ref[...][0:1, :] / N_NODES_
        row = i * BLK + lax.broadcasted_iota(jnp.int32, (BLK, 128), 0)
        dv = jnp.where(row < N_NODES_, hpre_ref[...] - mu, 0.0)
        s = jnp.sum(dv * dv, axis=0, keepdims=True)

        @pl.when(i == 0)
        def _():
            var_ref[...] = jnp.zeros_like(var_ref)
        var_ref[...] += s

    return pl.pallas_call(
        body,
        grid=(GRID,),
        in_specs=[
            pl.BlockSpec((BLK, 128), lambda i: (i, 0)),
            pl.BlockSpec((2, 128), lambda i: (0, 0)),
        ],
        out_specs=pl.BlockSpec((1, 128), lambda i: (0, 0)),
        out_shape=jax.ShapeDtypeStruct((1, 128), F32),
    )(hpre, st)


def _stage_post_row(hpre, st, varsum, g_row, be_row):
    def body(hpre_ref, st_ref, var_ref, g_ref, be_ref, out_ref, hT_ref):
        mu = st_ref[...][0:1, :] / N_NODES_
        var = var_ref[...] / N_NODES_
        h = jnp.maximum(
            (hpre_ref[...] - mu) / jnp.sqrt(var + EPS) * g_ref[...]
            + be_ref[...], 0.0)
        out_ref[...] = h
        hT_ref[...] = h.T

    return pl.pallas_call(
        body,
        grid=(GRID,),
        in_specs=[
            pl.BlockSpec((BLK, 128), lambda i: (i, 0)),
            pl.BlockSpec((2, 128), lambda i: (0, 0)),
            pl.BlockSpec((1, 128), lambda i: (0, 0)),
            pl.BlockSpec((1, 128), lambda i: (0, 0)),
            pl.BlockSpec((1, 128), lambda i: (0, 0)),
        ],
        out_specs=[
            pl.BlockSpec((BLK, 128), lambda i: (i, 0)),
            pl.BlockSpec((128, BLK), lambda i: (0, i)),
        ],
        out_shape=[
            jax.ShapeDtypeStruct((N_PAD, 128), F32),
            jax.ShapeDtypeStruct((128, N_PAD), F32),
        ],
    )(hpre, st, varsum, g_row, be_row)


def _stage_pre2(h_T, agg_T, Wa, ba_t, Wb, bb_t):
    def body(h_ref, agg_ref, wa_ref, ba_ref, wb_ref, bb_ref, hpre_ref, st_ref):
        i = pl.program_id(0)
        s_in = h_ref[...] + agg_ref[...]
        wa = wa_ref[...]
        wb = wb_ref[...]
        outs = []
        for k in range(N_COMS_):
            u = lax.dot_general(wa, s_in[k * COM_DIM_:(k + 1) * COM_DIM_, :],
                                (((0,), (0,)), ((), ())),
                                preferred_element_type=F32)
            u = jnp.maximum(u + ba_ref[0:COM_DIM_, :], 0.0)
            outs.append(lax.dot_general(wb, u, (((0,), (0,)), ((), ())),
                                        preferred_element_type=F32))
        hpre = jnp.concatenate(outs, axis=0) + bb_ref[...]
        hpre_ref[...] = hpre
        col = i * BLK + lax.broadcasted_iota(jnp.int32, (128, BLK), 1)
        hm = jnp.where(col < N_NODES_, hpre, 0.0)
        su = jnp.sum(hm, axis=1, keepdims=True)
        ss = jnp.sum(hm * hm, axis=1, keepdims=True)

        @pl.when(i == 0)
        def _():
            st_ref[...] = jnp.zeros_like(st_ref)
        st_ref[...] += jnp.concatenate([su, ss], axis=1)

    return pl.pallas_call(
        body,
        grid=(GRID,),
        in_specs=[
            pl.BlockSpec((128, BLK), lambda i: (0, i)),
            pl.BlockSpec((128, BLK), lambda i: (0, i)),
            pl.BlockSpec((COM_DIM_, COM_DIM_), lambda i: (0, 0)),
            pl.BlockSpec((128, 1), lambda i: (0, 0)),
            pl.BlockSpec((COM_DIM_, COM_DIM_), lambda i: (0, 0)),
            pl.BlockSpec((128, 1), lambda i: (0, 0)),
        ],
        out_specs=[
            pl.BlockSpec((128, BLK), lambda i: (0, i)),
            pl.BlockSpec((128, 2), lambda i: (0, 0)),
        ],
        out_shape=[
            jax.ShapeDtypeStruct((128, N_PAD), F32),
            jax.ShapeDtypeStruct((128, 2), F32),
        ],
    )(h_T, agg_T, Wa, ba_t, Wb, bb_t)


def _stage_var(hpre_T, st):
    def body(hpre_ref, st_ref, var_ref):
        i = pl.program_id(0)
        mu = st_ref[...][:, 0:1] / N_NODES_
        col = i * BLK + lax.broadcasted_iota(jnp.int32, (128, BLK), 1)
        dv = jnp.where(col < N_NODES_, hpre_ref[...] - mu, 0.0)
        s = jnp.sum(dv * dv, axis=1, keepdims=True)

        @pl.when(i == 0)
        def _():
            var_ref[...] = jnp.zeros_like(var_ref)
        var_ref[...] += s

    return pl.pallas_call(
        body,
        grid=(GRID,),
        in_specs=[
            pl.BlockSpec((128, BLK), lambda i: (0, i)),
            pl.BlockSpec((128, 2), lambda i: (0, 0)),
        ],
        out_specs=pl.BlockSpec((128, 1), lambda i: (0, 0)),
        out_shape=jax.ShapeDtypeStruct((128, 1), F32),
    )(hpre_T, st)


def _stage_post(hpre_T, st, varsum, g_t, be_t):
    def body(hpre_ref, st_ref, var_ref, g_ref, be_ref, out_ref):
        mu = st_ref[...][:, 0:1] / N_NODES_
        var = var_ref[...] / N_NODES_
        h = jnp.maximum(
            (hpre_ref[...] - mu) / jnp.sqrt(var + EPS) * g_ref[...]
            + be_ref[...], 0.0)
        out_ref[...] = h.T

    return pl.pallas_call(
        body,
        grid=(GRID,),
        in_specs=[
            pl.BlockSpec((128, BLK), lambda i: (0, i)),
            pl.BlockSpec((128, 2), lambda i: (0, 0)),
            pl.BlockSpec((128, 1), lambda i: (0, 0)),
            pl.BlockSpec((128, 1), lambda i: (0, 0)),
            pl.BlockSpec((128, 1), lambda i: (0, 0)),
        ],
        out_specs=pl.BlockSpec((BLK, 128), lambda i: (i, 0)),
        out_shape=jax.ShapeDtypeStruct((N_PAD, 128), F32),
    )(hpre_T, st, varsum, g_t, be_t)


def _sc_mesh():
    return plsc.VectorSubcoreMesh(core_axis_name="c", subcore_axis_name="s",
                                  num_cores=NC, num_subcores=NS)


def _zero_accs(accs):
    @pl.loop(0, L_SC // 16)
    def _(i):
        z = jnp.zeros((16,), F32)
        for a in accs:
            a[pl.ds(i * 16, 16)] = z


def _edge_pipeline(src_hbm, dst_hbm, w_hbm, w_offs, sbufs, dbufs, wbufs,
                   sems, ec, process):
    n_chunks = N_EDGES_ // ec

    def start(g, b):
        base = g * ec
        pltpu.async_copy(src_hbm.at[pl.ds(base, ec)], sbufs[b], sems[b])
        pltpu.async_copy(dst_hbm.at[pl.ds(base, ec)], dbufs[b], sems[b])
        for t, off in enumerate(w_offs):
            pltpu.async_copy(w_hbm.at[pl.ds(off + base, ec)], wbufs[t][b],
                             sems[b])

    def wait(b):
        pltpu.make_async_copy(src_hbm.at[pl.ds(0, ec)], sbufs[b],
                              sems[b]).wait()
        pltpu.make_async_copy(dst_hbm.at[pl.ds(0, ec)], dbufs[b],
                              sems[b]).wait()
        for t in range(len(w_offs)):
            pltpu.make_async_copy(w_hbm.at[pl.ds(0, ec)], wbufs[t][b],
                                  sems[b]).wait()

    start(0, 0)

    @pl.loop(0, n_chunks, step=2)
    def _(g):
        start(g + 1, 1)
        wait(0)
        process(0)

        @pl.when(g + 2 < n_chunks)
        def _():
            start(g + 2, 0)
        wait(1)
        process(1)


def _make_sc_full():
    EC = 800
    scratch = (
        [pltpu.VMEM((L_SC,), F32) for _ in range(4)]
        + [pltpu.VMEM((L_SC,), F32) for _ in range(8)]
        + [pltpu.VMEM((EC,), jnp.int32) for _ in range(4)]
        + [pltpu.VMEM((EC,), F32) for _ in range(4)]
        + [pltpu.SemaphoreType.DMA, pltpu.SemaphoreType.DMA]
    )

    @functools.partial(
        pl.kernel,
        out_type=jax.ShapeDtypeStruct((256 * N_PAD,), F32),
        mesh=_sc_mesh(),
        scratch_types=scratch,
        compiler_params=pltpu.CompilerParams(needs_layout_passes=False),
    )
    def seg(xT_hbm, src_hbm, dst_hbm, w2_hbm, agg_hbm, *refs):
        tabs = refs[0:4]
        accs = refs[4:12]
        sbufs = refs[12:14]
        dbufs = refs[14:16]
        wb = refs[16:20]
        wbufs = [wb[0:2], wb[2:4]]
        sems = refs[20:22]

        wid = lax.axis_index("s") * NC + lax.axis_index("c")

        for j in range(4):
            off = pl.multiple_of((4 * wid + j) * N_PAD, 8)
            pltpu.sync_copy(xT_hbm.at[pl.ds(off, L_SC)], tabs[j])
        _zero_accs(accs)

        def process(b):
            sb, db = sbufs[b], dbufs[b]

            @plsc.parallel_loop(0, EC // 16, unroll=8)
            def _(i):
                off = i * 16
                sidx = sb[pl.ds(off, 16)]
                didx = db[pl.ds(off, 16)]
                w0 = wbufs[0][b][pl.ds(off, 16)]
                w1 = wbufs[1][b][pl.ds(off, 16)]
                for j in range(4):
                    v = plsc.load_gather(tabs[j], [sidx])
                    plsc.addupdate_scatter(accs[j], [didx], v * w0)
                    plsc.addupdate_scatter(accs[4 + j], [didx], v * w1)

        _edge_pipeline(src_hbm, dst_hbm, w2_hbm, (0, N_EDGES_),
                       sbufs, dbufs, wbufs, sems, EC, process)

        for c2 in range(2):
            for j in range(4):
                off = pl.multiple_of((c2 * 128 + 4 * wid + j) * N_PAD, 8)
                pltpu.sync_copy(accs[4 * c2 + j],
                                agg_hbm.at[pl.ds(off, L_SC)])

    return seg


def _make_sc_perk():
    EC = 2000
    scratch = (
        [pltpu.VMEM((L_SC,), F32) for _ in range(4)]
        + [pltpu.VMEM((L_SC,), F32) for _ in range(4)]
        + [pltpu.VMEM((EC,), jnp.int32) for _ in range(4)]
        + [pltpu.VMEM((EC,), F32) for _ in range(8)]
        + [pltpu.SemaphoreType.DMA, pltpu.SemaphoreType.DMA]
    )

    @functools.partial(
        pl.kernel,
        out_type=jax.ShapeDtypeStruct((128 * N_PAD,), F32),
        mesh=_sc_mesh(),
        scratch_types=scratch,
        compiler_params=pltpu.CompilerParams(needs_layout_passes=False),
    )
    def seg(hT_hbm, src_hbm, dst_hbm, w_hbm, agg_hbm, *refs):
        tabs = refs[0:4]
        accs = refs[4:8]
        sbufs = refs[8:10]
        dbufs = refs[10:12]
        wb = refs[12:20]
        wbufs = [wb[0:2], wb[2:4], wb[4:6], wb[6:8]]
        sems = refs[20:22]

        wid = lax.axis_index("s") * NC + lax.axis_index("c")

        for k in range(N_COMS_):
            off = pl.multiple_of((k * COM_DIM_ + wid) * N_PAD, 8)
            pltpu.sync_copy(hT_hbm.at[pl.ds(off, L_SC)], tabs[k])
        _zero_accs(accs)

        def process(b):
            sb, db = sbufs[b], dbufs[b]

            @plsc.parallel_loop(0, EC // 16, unroll=8)
            def _(i):
                off = i * 16
                sidx = sb[pl.ds(off, 16)]
                didx = db[pl.ds(off, 16)]
                for k in range(N_COMS_):
                    v = plsc.load_gather(tabs[k], [sidx])
                    wk = wbufs[k][b][pl.ds(off, 16)]
                    plsc.addupdate_scatter(accs[k], [didx], v * wk)

        _edge_pipeline(src_hbm, dst_hbm, w_hbm,
                       tuple(k * N_EDGES_ for k in range(N_COMS_)),
                       sbufs, dbufs, wbufs, sems, EC, process)

        for k in range(N_COMS_):
            off = pl.multiple_of((k * COM_DIM_ + wid) * N_PAD, 8)
            pltpu.sync_copy(accs[k], agg_hbm.at[pl.ds(off, L_SC)])

    return seg


_sc_seg_full = _make_sc_full()
_sc_seg_perk = _make_sc_perk()


def kernel(x, edge_index, edge_weight_list, W_enc, b_enc,
           W0a, b0a, W0b, b0b, g0, be0,
           W1a, b1a, W1b, b1b, g1, be1):
    src = edge_index[0].astype(jnp.int32)
    dst = edge_index[1].astype(jnp.int32)
    wflat = edge_weight_list.astype(F32).reshape(-1)
    xp = jnp.pad(x.astype(F32), ((0, N_PAD - N_NODES_), (0, 0)))

    enc_p, xT = _stage0(xp, W_enc, b_enc.reshape(1, -1))

    xT_flat = xT.reshape(-1)
    agg0a = _sc_seg_full(xT_flat, src, dst,
                         wflat[:2 * N_EDGES_]).reshape(256, N_PAD)
    agg0b = _sc_seg_full(xT_flat, src, dst,
                         wflat[2 * N_EDGES_:]).reshape(256, N_PAD)
    h1pre, st1 = _stage1(xp, agg0a, agg0b, W0a, b0a.reshape(1, -1),
                         W0b, b0b.reshape(1, -1))
    vs1 = _stage_var_row(h1pre, st1)
    out1_p, h1T = _stage_post_row(h1pre, st1, vs1,
                                  jnp.tile(g0, N_COMS_).reshape(1, -1),
                                  jnp.tile(be0, N_COMS_).reshape(1, -1))

    agg1 = _sc_seg_perk(h1T.reshape(-1), src, dst,
                        wflat).reshape(128, N_PAD)
    h2pre, st2 = _stage_pre2(h1T, agg1, W1a,
                             jnp.tile(b1a, N_COMS_).reshape(-1, 1),
                             W1b, jnp.tile(b1b, N_COMS_).reshape(-1, 1))
    vs2 = _stage_var(h2pre, st2)
    out2_p = _stage_post(h2pre, st2, vs2,
                         jnp.tile(g1, N_COMS_).reshape(-1, 1),
                         jnp.tile(be1, N_COMS_).reshape(-1, 1))

    return (enc_p[:N_NODES_], out1_p[:N_NODES_], out2_p[:N_NODES_])
